# Initial kernel scaffold; baseline (speedup 1.0000x reference)
import jax
import jax.numpy as jnp
from jax.experimental import pallas as pl


def kernel(x, edge_index, batch, params):
    raise NotImplementedError("write your pallas kernel here")


# speedup vs baseline: 143.4582x; 143.4582x over previous
import functools

import jax
import jax.numpy as jnp
import numpy as np
from jax import lax
from jax.experimental import pallas as pl
from jax.experimental.pallas import tpu as pltpu
from jax.experimental.pallas import tpu_sc as plsc

H = 8
C = 8
HC = 64
NEG_SLOPE = 0.2
G = 64
NC, NS = 2, 16
NT = NC * NS
CH = 128
ROWB = 512
ACC_W = 80

_PERM = np.array([32 * (v >> 1) + 8 * (m >> 2) + 2 * (m & 3) + (v & 1)
                  for v in range(4) for m in range(16)])
_STORE = np.empty(64, np.int64)
for _h in range(2):
    for _m in range(16):
        _STORE[32 * _h + 2 * _m] = _PERM[32 * _h + _m]
        _STORE[32 * _h + 2 * _m + 1] = _PERM[32 * _h + 16 + _m]
_CPERM = np.array([2 * (cp & 3) + (cp >> 2) for cp in range(8)])


def _expand_den(den):
    r = den.shape[0]
    d = den.reshape(r, 2, 1, 4, 1)
    return jnp.broadcast_to(d, (r, 2, 2, 4, 4)).reshape(r, HC)


def _mm_body(x_ref, wl_ref, wr_ref, bl_ref, br_ref, xl_ref, xr_ref):
    x = x_ref[...]
    xl = jnp.dot(x, wl_ref[...], preferred_element_type=jnp.float32) + bl_ref[...]
    xr = jnp.dot(x, wr_ref[...], preferred_element_type=jnp.float32) + br_ref[...]
    xl_ref[...] = xl.astype(jnp.bfloat16)
    xr_ref[...] = xr.astype(jnp.bfloat16)


def _mm(x_pad, wl, wr, bl, br):
    n_pad, d = x_pad.shape
    return pl.pallas_call(
        _mm_body,
        grid=(n_pad // ROWB,),
        in_specs=[
            pl.BlockSpec((ROWB, d), lambda j: (j, 0)),
            pl.BlockSpec((d, HC), lambda j: (0, 0)),
            pl.BlockSpec((d, HC), lambda j: (0, 0)),
            pl.BlockSpec((1, HC), lambda j: (0, 0)),
            pl.BlockSpec((1, HC), lambda j: (0, 0)),
        ],
        out_specs=[
            pl.BlockSpec((ROWB, HC), lambda j: (j, 0)),
            pl.BlockSpec((ROWB, HC), lambda j: (j, 0)),
        ],
        out_shape=[jax.ShapeDtypeStruct((n_pad, HC), jnp.bfloat16)] * 2,
    )(x_pad, wl, wr, bl.reshape(1, HC), br.reshape(1, HC))


def _epi_mm_body(slab_ref, bias_ref, wl_ref, wr_ref, bl_ref, br_ref,
                 xl_ref, xr_ref):
    s0 = slab_ref[0]
    s1 = slab_ref[1]
    num = s0[:, :64] + s1[:, :64]
    den = s0[:, 64:72] + s1[:, 64:72]
    den_exp = _expand_den(den)
    h = num / (den_exp + 1e-16) + bias_ref[...]
    h = jnp.where(h > 0, h, jnp.exp(h) - 1.0)
    xl = jnp.dot(h, wl_ref[...], preferred_element_type=jnp.float32) + bl_ref[...]
    xr = jnp.dot(h, wr_ref[...], preferred_element_type=jnp.float32) + br_ref[...]
    xl_ref[...] = xl.astype(jnp.bfloat16)
    xr_ref[...] = xr.astype(jnp.bfloat16)


def _epi_mm(slab, bias, wl, wr, bl, br):
    n_pad = slab.shape[1]
    return pl.pallas_call(
        _epi_mm_body,
        grid=(n_pad // ROWB,),
        in_specs=[
            pl.BlockSpec((2, ROWB, ACC_W), lambda j: (0, j, 0)),
            pl.BlockSpec((1, HC), lambda j: (0, 0)),
            pl.BlockSpec((HC, HC), lambda j: (0, 0)),
            pl.BlockSpec((HC, HC), lambda j: (0, 0)),
            pl.BlockSpec((1, HC), lambda j: (0, 0)),
            pl.BlockSpec((1, HC), lambda j: (0, 0)),
        ],
        out_specs=[
            pl.BlockSpec((ROWB, HC), lambda j: (j, 0)),
            pl.BlockSpec((ROWB, HC), lambda j: (j, 0)),
        ],
        out_shape=[jax.ShapeDtypeStruct((n_pad, HC), jnp.bfloat16)] * 2,
    )(slab, bias.reshape(1, HC), wl, wr, bl.reshape(1, HC), br.reshape(1, HC))


def _final_body(slab_ref, bias_ref, batch_ref, hw_ref, hb_ref, out_ref, acc_ref):
    j = pl.program_id(0)
    s0 = slab_ref[0]
    s1 = slab_ref[1]
    num = s0[:, :64] + s1[:, :64]
    den = s0[:, 64:72] + s1[:, 64:72]
    den_exp = _expand_den(den)
    w = num / (den_exp + 1e-16)
    m = None
    for q2 in range(2):
        for jj in range(4):
            b0 = w[:, 32 * q2 + 4 * jj:32 * q2 + 4 * jj + 4]
            b1 = w[:, 32 * q2 + 16 + 4 * jj:32 * q2 + 16 + 4 * jj + 4]
            blk = jnp.concatenate([b0, b1], axis=1)
            m = blk if m is None else m + blk
    m = m * (1.0 / H) + bias_ref[...]
    hfin = jnp.where(m > 0, m, jnp.exp(m) - 1.0)
    bb = batch_ref[0]
    gids = lax.broadcasted_iota(jnp.int32, (G, 1), 0)
    oh = (gids == bb).astype(jnp.float32)
    contrib = jnp.dot(oh, hfin, preferred_element_type=jnp.float32)

    @pl.when(j == 0)
    def _():
        acc_ref[...] = contrib

    @pl.when(j > 0)
    def _():
        acc_ref[...] = acc_ref[...] + contrib

    @pl.when(j == pl.num_programs(0) - 1)
    def _():
        out_ref[...] = jnp.dot(acc_ref[...], hw_ref[...],
                               preferred_element_type=jnp.float32) + hb_ref[...]


def _final(slab, bias, batch3, head_w, head_b):
    n_pad = slab.shape[1]
    return pl.pallas_call(
        _final_body,
        grid=(n_pad // ROWB,),
        in_specs=[
            pl.BlockSpec((2, ROWB, ACC_W), lambda j: (0, j, 0)),
            pl.BlockSpec((1, C), lambda j: (0, 0)),
            pl.BlockSpec((1, 1, ROWB), lambda j: (j, 0, 0)),
            pl.BlockSpec((C, 1), lambda j: (0, 0)),
            pl.BlockSpec((1, 1), lambda j: (0, 0)),
        ],
        out_specs=pl.BlockSpec((G, 1), lambda j: (0, 0)),
        out_shape=jax.ShapeDtypeStruct((G, 1), jnp.float32),
        scratch_shapes=[pltpu.VMEM((G, C), jnp.float32)],
    )(slab, bias.reshape(1, C), batch3, head_w, head_b.reshape(1, 1))


def _take16(v, idx):
    dn = lax.GatherDimensionNumbers(
        offset_dims=(), collapsed_slice_dims=(0,), start_index_map=(0,))
    return lax.gather(v, idx[:, None], dn, slice_sizes=(1,),
                      mode=lax.GatherScatterMode.PROMISE_IN_BOUNDS)


def _split_bf16(ref, e, half):
    w = ref[e, pl.ds(16 * half, 16)]
    a = lax.bitcast_convert_type(w << 16, jnp.float32)
    b = lax.bitcast_convert_type(w & np.int32(-65536), jnp.float32)
    return a, b


def _edge_body(src_hbm, dst_hbm, xl_hbm, xr_hbm, att_hbm, out_hbm,
               srcv, dstv, xlr0, xrr0, xlr1, xrr1, pay0, pay1, attv,
               accum, sem0, sem1, ssem0, ssem1, *, k0, k1):
    cid = lax.axis_index("c")
    sid = lax.axis_index("s")
    n_pad = accum.shape[0]
    rows_per_tile = n_pad // NS

    def zero_row(r, carry):
        for k in range(ACC_W // 16):
            pay0[r, pl.ds(16 * k, 16)] = jnp.zeros((16,), jnp.float32)
        return carry
    lax.fori_loop(0, CH, zero_row, 0)

    r0 = sid * rows_per_tile
    nfull = rows_per_tile // CH
    rem = rows_per_tile - nfull * CH
    for i in range(nfull):
        pltpu.sync_copy(pay0, accum.at[pl.ds(r0 + i * CH, CH)])
    if rem:
        pltpu.sync_copy(pay0.at[pl.ds(0, rem)], accum.at[pl.ds(r0 + nfull * CH, rem)])

    pltpu.sync_copy(att_hbm, attv)

    @pl.when(cid == 0)
    def _():
        pltpu.sync_copy(src_hbm.at[pl.ds(sid * k0, k0)], srcv.at[pl.ds(0, k0)])
        pltpu.sync_copy(dst_hbm.at[pl.ds(sid * k0, k0)], dstv.at[pl.ds(0, k0)])

    @pl.when(cid == 1)
    def _():
        b = NS * k0 + sid * k1
        pltpu.sync_copy(src_hbm.at[pl.ds(b, k1)], srcv.at[pl.ds(0, k1)])
        pltpu.sync_copy(dst_hbm.at[pl.ds(b, k1)], dstv.at[pl.ds(0, k1)])

    half_chunks = jnp.where(cid == 0, k0 // 2, k1 // 2)
    plsc.subcore_barrier()

    lane = lax.iota(jnp.int32, 16)
    idx4 = (lane & 3) * 4
    hgrp = lane >> 2
    mhalf = [(1 - jnp.minimum(jnp.abs(hgrp - hf), 1)).astype(jnp.float32)
             for hf in range(2)]
    atts = [attv[pl.ds(16 * v, 16)] for v in range(4)]

    def make_edge_one(xlr, xrr, pay):
        def edge_one(e, carry=None):
            p = jnp.zeros((16,), jnp.float32)
            for hf in range(2):
                la, lb = _split_bf16(xlr, e, hf)
                ra, rb = _split_bf16(xrr, e, hf)
                za = la + ra
                zb = lb + rb
                zla = jnp.where(za >= 0, za, NEG_SLOPE * za)
                zlb = jnp.where(zb >= 0, zb, NEG_SLOPE * zb)
                u = zla * atts[2 * hf] + zlb * atts[2 * hf + 1]
                for k in (1, 2):
                    u = u + _take16(u, lane ^ k)
                ex = jnp.exp(u)
                pay[e, pl.ds(32 * hf, 16)] = la * ex
                pay[e, pl.ds(32 * hf + 16, 16)] = lb * ex
                p = p + _take16(ex, idx4) * mhalf[hf]
            pay[e, pl.ds(64, 16)] = p
        return edge_one

    edge_fns = (make_edge_one(xlr0, xrr0, pay0), make_edge_one(xlr1, xrr1, pay1))
    bufs = ((xlr0, xrr0, sem0), (xlr1, xrr1, sem1))
    pays = ((pay0, ssem0), (pay1, ssem1))

    def gather_pair(j, b):
        xlr, xrr, sem = bufs[b]
        pltpu.make_async_copy(xl_hbm.at[srcv.at[j]], xlr, sem).start()
        pltpu.make_async_copy(xr_hbm.at[dstv.at[j]], xrr, sem).start()

    def wait_pair(j, b):
        xlr, xrr, sem = bufs[b]
        pltpu.make_async_copy(xl_hbm.at[srcv.at[j]], xlr, sem).wait()
        pltpu.make_async_copy(xr_hbm.at[dstv.at[j]], xrr, sem).wait()

    def scatter_start(j, b):
        pay, ssem = pays[b]
        pltpu.async_copy(pay, accum.at[dstv.at[j]], ssem, add=True)

    def scatter_wait(b):
        pay, ssem = pays[b]
        pltpu.make_async_copy(pay, accum.at[dstv.at[0]], ssem).wait()

    gather_pair(0, 0)

    def chunk_pair(t, carry):
        j0 = t * 2
        gather_pair(j0 + 1, 1)
        wait_pair(j0, 0)

        @pl.when(t > 0)
        def _():
            scatter_wait(0)
        plsc.parallel_loop(0, CH, unroll=4)(edge_fns[0])
        scatter_start(j0, 0)
        @pl.when(t < half_chunks - 1)
        def _():
            gather_pair(j0 + 2, 0)
        wait_pair(j0 + 1, 1)

        @pl.when(t > 0)
        def _():
            scatter_wait(1)
        plsc.parallel_loop(0, CH, unroll=4)(edge_fns[1])
        scatter_start(j0 + 1, 1)
        return carry

    lax.fori_loop(0, half_chunks, chunk_pair, 0)
    scatter_wait(0)
    scatter_wait(1)

    plsc.subcore_barrier()
    pltpu.sync_copy(accum.at[pl.ds(r0, rows_per_tile)],
                    out_hbm.at[cid, pl.ds(r0, rows_per_tile)])


def _edge_stage(src3, dst3, xl, xr, attflat, n_pad, k0, k1):
    n_pad_rows = xl.shape[0]
    xl = lax.bitcast_convert_type(xl.reshape(n_pad_rows, HC // 2, 2), jnp.int32)
    xr = lax.bitcast_convert_type(xr.reshape(n_pad_rows, HC // 2, 2), jnp.int32)
    kmax = max(k0, k1)
    kern = pl.kernel(
        functools.partial(_edge_body, k0=k0, k1=k1),
        out_type=jax.ShapeDtypeStruct((NC, n_pad, ACC_W), jnp.float32),
        mesh=plsc.VectorSubcoreMesh(core_axis_name="c", subcore_axis_name="s",
                                    num_cores=NC, num_subcores=NS),
        compiler_params=pltpu.CompilerParams(use_tc_tiling_on_sc=False),
        scratch_types=[
            pltpu.VMEM((kmax, CH), jnp.int32),
            pltpu.VMEM((kmax, CH), jnp.int32),
            pltpu.VMEM((CH, HC // 2), jnp.int32),
            pltpu.VMEM((CH, HC // 2), jnp.int32),
            pltpu.VMEM((CH, HC // 2), jnp.int32),
            pltpu.VMEM((CH, HC // 2), jnp.int32),
            pltpu.VMEM((CH, ACC_W), jnp.float32),
            pltpu.VMEM((CH, ACC_W), jnp.float32),
            pltpu.VMEM((HC,), jnp.float32),
            pltpu.VMEM_SHARED((n_pad, ACC_W), jnp.float32),
            pltpu.SemaphoreType.DMA,
            pltpu.SemaphoreType.DMA,
            pltpu.SemaphoreType.DMA,
            pltpu.SemaphoreType.DMA,
        ],
    )
    return kern(src3, dst3, xl, xr, attflat)


def kernel(x, edge_index, batch, params):
    n = x.shape[0]
    e = edge_index.shape[1]
    n_pad = ((n + 1 + ROWB - 1) // ROWB) * ROWB
    assert n_pad % (NS * 8) == 0

    src = edge_index[0]
    dst = edge_index[1]
    per_tile = (e + NT * CH - 1) // (NT * CH)
    per_tile += per_tile % 2
    k0 = per_tile
    k1 = per_tile
    n_rows = NS * (k0 + k1)
    e_pad = n_rows * CH
    src3 = jnp.concatenate(
        [src, jnp.zeros((e_pad - e,), jnp.int32)]).reshape(n_rows, CH)
    dst3 = jnp.concatenate(
        [dst, jnp.full((e_pad - e,), n, jnp.int32)]).reshape(n_rows, CH)

    x_pad = jnp.pad(x, ((0, n_pad - n), (0, 0)))
    batch3 = jnp.pad(batch, (0, n_pad - n),
                     constant_values=G).reshape(n_pad // ROWB, 1, ROWB)

    layers = params["layers"]
    l0 = layers[0]
    xl, xr = _mm(x_pad, l0["Wl"][:, _STORE], l0["Wr"][:, _STORE],
                 l0["bl"][_STORE], l0["br"][_STORE])
    for li in range(3):
        slab = _edge_stage(src3, dst3, xl, xr,
                           layers[li]["att"].reshape(HC)[_PERM], n_pad, k0, k1)
        nxt = layers[li + 1]
        xl, xr = _epi_mm(slab, layers[li]["bias"][_PERM],
                         nxt["Wl"][_PERM][:, _STORE], nxt["Wr"][_PERM][:, _STORE],
                         nxt["bl"][_STORE], nxt["br"][_STORE])
    slab = _edge_stage(src3, dst3, xl, xr,
                       layers[3]["att"].reshape(HC)[_PERM], n_pad, k0, k1)
    return _final(slab, layers[3]["bias"][_CPERM], batch3,
                  params["head_w"][_CPERM, :], params["head_b"])


# speedup vs baseline: 147.4080x; 1.0275x over previous
import functools

import jax
import jax.numpy as jnp
import numpy as np
from jax import lax
from jax.experimental import pallas as pl
from jax.experimental.pallas import tpu as pltpu
from jax.experimental.pallas import tpu_sc as plsc

H = 8
C = 8
HC = 64
NEG_SLOPE = 0.2
G = 64
NC, NS = 2, 16
NT = NC * NS
CH = 128
ROWB = 512
ACC_W = 80

_PERM = np.array([32 * (v >> 1) + 8 * (m >> 2) + 2 * (m & 3) + (v & 1)
                  for v in range(4) for m in range(16)])
_STORE = np.empty(64, np.int64)
for _h in range(2):
    for _m in range(16):
        _STORE[32 * _h + 2 * _m] = _PERM[32 * _h + _m]
        _STORE[32 * _h + 2 * _m + 1] = _PERM[32 * _h + 16 + _m]
_CPERM = np.array([2 * (cp & 3) + (cp >> 2) for cp in range(8)])


def _expand_den(den):
    r = den.shape[0]
    d = den.reshape(r, 2, 1, 4, 1)
    return jnp.broadcast_to(d, (r, 2, 2, 4, 4)).reshape(r, HC)


def _mm_body(x_ref, wl_ref, wr_ref, bl_ref, br_ref, xl_ref, xr_ref):
    x = x_ref[...]
    xl = jnp.dot(x, wl_ref[...], preferred_element_type=jnp.float32) + bl_ref[...]
    xr = jnp.dot(x, wr_ref[...], preferred_element_type=jnp.float32) + br_ref[...]
    xl_ref[...] = xl.astype(jnp.bfloat16)
    xr_ref[...] = xr.astype(jnp.bfloat16)


def _mm(x_pad, wl, wr, bl, br):
    n_pad, d = x_pad.shape
    return pl.pallas_call(
        _mm_body,
        grid=(n_pad // ROWB,),
        in_specs=[
            pl.BlockSpec((ROWB, d), lambda j: (j, 0)),
            pl.BlockSpec((d, HC), lambda j: (0, 0)),
            pl.BlockSpec((d, HC), lambda j: (0, 0)),
            pl.BlockSpec((1, HC), lambda j: (0, 0)),
            pl.BlockSpec((1, HC), lambda j: (0, 0)),
        ],
        out_specs=[
            pl.BlockSpec((ROWB, HC), lambda j: (j, 0)),
            pl.BlockSpec((ROWB, HC), lambda j: (j, 0)),
        ],
        out_shape=[jax.ShapeDtypeStruct((n_pad, HC), jnp.bfloat16)] * 2,
    )(x_pad, wl, wr, bl.reshape(1, HC), br.reshape(1, HC))


def _epi_mm_body(slab_ref, bias_ref, wl_ref, wr_ref, bl_ref, br_ref,
                 xl_ref, xr_ref):
    s0 = slab_ref[0]
    s1 = slab_ref[1]
    num = s0[:, :64] + s1[:, :64]
    den = s0[:, 64:72] + s1[:, 64:72]
    den_exp = _expand_den(den)
    h = num / (den_exp + 1e-16) + bias_ref[...]
    h = jnp.where(h > 0, h, jnp.exp(h) - 1.0)
    xl = jnp.dot(h, wl_ref[...], preferred_element_type=jnp.float32) + bl_ref[...]
    xr = jnp.dot(h, wr_ref[...], preferred_element_type=jnp.float32) + br_ref[...]
    xl_ref[...] = xl.astype(jnp.bfloat16)
    xr_ref[...] = xr.astype(jnp.bfloat16)


def _epi_mm(slab, bias, wl, wr, bl, br):
    n_pad = slab.shape[1]
    return pl.pallas_call(
        _epi_mm_body,
        grid=(n_pad // ROWB,),
        in_specs=[
            pl.BlockSpec((2, ROWB, ACC_W), lambda j: (0, j, 0)),
            pl.BlockSpec((1, HC), lambda j: (0, 0)),
            pl.BlockSpec((HC, HC), lambda j: (0, 0)),
            pl.BlockSpec((HC, HC), lambda j: (0, 0)),
            pl.BlockSpec((1, HC), lambda j: (0, 0)),
            pl.BlockSpec((1, HC), lambda j: (0, 0)),
        ],
        out_specs=[
            pl.BlockSpec((ROWB, HC), lambda j: (j, 0)),
            pl.BlockSpec((ROWB, HC), lambda j: (j, 0)),
        ],
        out_shape=[jax.ShapeDtypeStruct((n_pad, HC), jnp.bfloat16)] * 2,
    )(slab, bias.reshape(1, HC), wl, wr, bl.reshape(1, HC), br.reshape(1, HC))


def _final_body(slab_ref, bias_ref, batch_ref, hw_ref, hb_ref, out_ref, acc_ref):
    j = pl.program_id(0)
    s0 = slab_ref[0]
    s1 = slab_ref[1]
    num = s0[:, :64] + s1[:, :64]
    den = s0[:, 64:72] + s1[:, 64:72]
    den_exp = _expand_den(den)
    w = num / (den_exp + 1e-16)
    m = None
    for q2 in range(2):
        for jj in range(4):
            b0 = w[:, 32 * q2 + 4 * jj:32 * q2 + 4 * jj + 4]
            b1 = w[:, 32 * q2 + 16 + 4 * jj:32 * q2 + 16 + 4 * jj + 4]
            blk = jnp.concatenate([b0, b1], axis=1)
            m = blk if m is None else m + blk
    m = m * (1.0 / H) + bias_ref[...]
    hfin = jnp.where(m > 0, m, jnp.exp(m) - 1.0)
    bb = batch_ref[0]
    gids = lax.broadcasted_iota(jnp.int32, (G, 1), 0)
    oh = (gids == bb).astype(jnp.float32)
    contrib = jnp.dot(oh, hfin, preferred_element_type=jnp.float32)

    @pl.when(j == 0)
    def _():
        acc_ref[...] = contrib

    @pl.when(j > 0)
    def _():
        acc_ref[...] = acc_ref[...] + contrib

    @pl.when(j == pl.num_programs(0) - 1)
    def _():
        out_ref[...] = jnp.dot(acc_ref[...], hw_ref[...],
                               preferred_element_type=jnp.float32) + hb_ref[...]


def _final(slab, bias, batch3, head_w, head_b):
    n_pad = slab.shape[1]
    return pl.pallas_call(
        _final_body,
        grid=(n_pad // ROWB,),
        in_specs=[
            pl.BlockSpec((2, ROWB, ACC_W), lambda j: (0, j, 0)),
            pl.BlockSpec((1, C), lambda j: (0, 0)),
            pl.BlockSpec((1, 1, ROWB), lambda j: (j, 0, 0)),
            pl.BlockSpec((C, 1), lambda j: (0, 0)),
            pl.BlockSpec((1, 1), lambda j: (0, 0)),
        ],
        out_specs=pl.BlockSpec((G, 1), lambda j: (0, 0)),
        out_shape=jax.ShapeDtypeStruct((G, 1), jnp.float32),
        scratch_shapes=[pltpu.VMEM((G, C), jnp.float32)],
    )(slab, bias.reshape(1, C), batch3, head_w, head_b.reshape(1, 1))


def _take16(v, idx):
    dn = lax.GatherDimensionNumbers(
        offset_dims=(), collapsed_slice_dims=(0,), start_index_map=(0,))
    return lax.gather(v, idx[:, None], dn, slice_sizes=(1,),
                      mode=lax.GatherScatterMode.PROMISE_IN_BOUNDS)


def _split_bf16(ref, e, half):
    w = ref[e, pl.ds(16 * half, 16)]
    a = lax.bitcast_convert_type(w << 16, jnp.float32)
    b = lax.bitcast_convert_type(w & np.int32(-65536), jnp.float32)
    return a, b


def _edge_body(src_hbm, dst_hbm, xl_hbm, xr_hbm, att_hbm, out_hbm,
               srcv, dstv, xlr0, xrr0, xlr1, xrr1, pay0, pay1, attv,
               accum, sem0, sem1, ssem0, ssem1, *, k0, k1):
    cid = lax.axis_index("c")
    sid = lax.axis_index("s")
    n_pad = accum.shape[0]
    rows_per_tile = n_pad // NS

    def zero_row(r, carry):
        for k in range(ACC_W // 16):
            pay0[r, pl.ds(16 * k, 16)] = jnp.zeros((16,), jnp.float32)
        return carry
    lax.fori_loop(0, CH, zero_row, 0)

    r0 = sid * rows_per_tile
    nfull = rows_per_tile // CH
    rem = rows_per_tile - nfull * CH
    for i in range(nfull):
        pltpu.sync_copy(pay0, accum.at[pl.ds(r0 + i * CH, CH)])
    if rem:
        pltpu.sync_copy(pay0.at[pl.ds(0, rem)], accum.at[pl.ds(r0 + nfull * CH, rem)])

    pltpu.sync_copy(att_hbm, attv)

    @pl.when(cid == 0)
    def _():
        pltpu.sync_copy(src_hbm.at[pl.ds(sid * k0, k0)], srcv.at[pl.ds(0, k0)])
        pltpu.sync_copy(dst_hbm.at[pl.ds(sid * k0, k0)], dstv.at[pl.ds(0, k0)])

    @pl.when(cid == 1)
    def _():
        b = NS * k0 + sid * k1
        pltpu.sync_copy(src_hbm.at[pl.ds(b, k1)], srcv.at[pl.ds(0, k1)])
        pltpu.sync_copy(dst_hbm.at[pl.ds(b, k1)], dstv.at[pl.ds(0, k1)])

    half_chunks = jnp.where(cid == 0, k0 // 2, k1 // 2)
    plsc.subcore_barrier()

    lane = lax.iota(jnp.int32, 16)
    idx4 = (lane & 3) * 4
    hgrp = lane >> 2
    mhalf = [(1 - jnp.minimum(jnp.abs(hgrp - hf), 1)).astype(jnp.float32)
             for hf in range(2)]
    atts = [attv[pl.ds(16 * v, 16)] for v in range(4)]

    def make_edge_one(xlr, xrr, pay):
        def edge_one(e, carry=None):
            p = jnp.zeros((16,), jnp.float32)
            for hf in range(2):
                la, lb = _split_bf16(xlr, e, hf)
                ra, rb = _split_bf16(xrr, e, hf)
                za = la + ra
                zb = lb + rb
                zla = jnp.where(za >= 0, za, NEG_SLOPE * za)
                zlb = jnp.where(zb >= 0, zb, NEG_SLOPE * zb)
                u = zla * atts[2 * hf] + zlb * atts[2 * hf + 1]
                for k in (1, 2):
                    u = u + _take16(u, lane ^ k)
                ex = jnp.exp(u)
                pay[e, pl.ds(32 * hf, 16)] = la * ex
                pay[e, pl.ds(32 * hf + 16, 16)] = lb * ex
                p = p + _take16(ex, idx4) * mhalf[hf]
            pay[e, pl.ds(64, 16)] = p
        return edge_one

    edge_fns = (make_edge_one(xlr0, xrr0, pay0), make_edge_one(xlr1, xrr1, pay1))
    bufs = ((xlr0, xrr0, sem0), (xlr1, xrr1, sem1))
    pays = ((pay0, ssem0), (pay1, ssem1))

    def gather_pair(j, b):
        xlr, xrr, sem = bufs[b]
        pltpu.make_async_copy(xl_hbm.at[srcv.at[j]], xlr, sem).start()
        pltpu.make_async_copy(xr_hbm.at[dstv.at[j]], xrr, sem).start()

    def wait_pair(j, b):
        xlr, xrr, sem = bufs[b]
        pltpu.make_async_copy(xl_hbm.at[srcv.at[j]], xlr, sem).wait()
        pltpu.make_async_copy(xr_hbm.at[dstv.at[j]], xrr, sem).wait()

    def scatter_start(j, b):
        pay, ssem = pays[b]
        pltpu.async_copy(pay, accum.at[dstv.at[j]], ssem, add=True)

    def scatter_wait(b):
        pay, ssem = pays[b]
        pltpu.make_async_copy(pay, accum.at[dstv.at[0]], ssem).wait()

    gather_pair(0, 0)

    def chunk_pair(t, carry):
        j0 = t * 2
        gather_pair(j0 + 1, 1)
        wait_pair(j0, 0)

        @pl.when(t > 0)
        def _():
            scatter_wait(0)
        plsc.parallel_loop(0, CH, unroll=4)(edge_fns[0])
        scatter_start(j0, 0)
        @pl.when(t < half_chunks - 1)
        def _():
            gather_pair(j0 + 2, 0)
        wait_pair(j0 + 1, 1)

        @pl.when(t > 0)
        def _():
            scatter_wait(1)
        plsc.parallel_loop(0, CH, unroll=4)(edge_fns[1])
        scatter_start(j0 + 1, 1)
        return carry

    lax.fori_loop(0, half_chunks, chunk_pair, 0)
    scatter_wait(0)
    scatter_wait(1)

    plsc.subcore_barrier()
    pltpu.sync_copy(accum.at[pl.ds(r0, rows_per_tile)],
                    out_hbm.at[cid, pl.ds(r0, rows_per_tile)])


def _edge_stage(src3, dst3, xl, xr, attflat, n_pad, k0, k1):
    n_pad_rows = xl.shape[0]
    xl = lax.bitcast_convert_type(xl.reshape(n_pad_rows, HC // 2, 2), jnp.int32)
    xr = lax.bitcast_convert_type(xr.reshape(n_pad_rows, HC // 2, 2), jnp.int32)
    kmax = max(k0, k1)
    kern = pl.kernel(
        functools.partial(_edge_body, k0=k0, k1=k1),
        out_type=jax.ShapeDtypeStruct((NC, n_pad, ACC_W), jnp.float32),
        mesh=plsc.VectorSubcoreMesh(core_axis_name="c", subcore_axis_name="s",
                                    num_cores=NC, num_subcores=NS),
        compiler_params=pltpu.CompilerParams(use_tc_tiling_on_sc=False),
        scratch_types=[
            pltpu.VMEM((kmax, CH), jnp.int32),
            pltpu.VMEM((kmax, CH), jnp.int32),
            pltpu.VMEM((CH, HC // 2), jnp.int32),
            pltpu.VMEM((CH, HC // 2), jnp.int32),
            pltpu.VMEM((CH, HC // 2), jnp.int32),
            pltpu.VMEM((CH, HC // 2), jnp.int32),
            pltpu.VMEM((CH, ACC_W), jnp.float32),
            pltpu.VMEM((CH, ACC_W), jnp.float32),
            pltpu.VMEM((HC,), jnp.float32),
            pltpu.VMEM_SHARED((n_pad, ACC_W), jnp.float32),
            pltpu.SemaphoreType.DMA,
            pltpu.SemaphoreType.DMA,
            pltpu.SemaphoreType.DMA,
            pltpu.SemaphoreType.DMA,
        ],
    )
    return kern(src3, dst3, xl, xr, attflat)


def kernel(x, edge_index, batch, params):
    n = x.shape[0]
    e = edge_index.shape[1]
    n_pad = ((n + 1 + ROWB - 1) // ROWB) * ROWB
    assert n_pad % (NS * 8) == 0

    src = edge_index[0]
    dst = edge_index[1]
    per_tile = (e + NT * CH - 1) // (NT * CH)
    per_tile += per_tile % 2
    k0 = (per_tile * 2) * 13 // 20
    k0 += k0 % 2
    k1 = per_tile * 2 - k0
    n_rows = NS * (k0 + k1)
    e_pad = n_rows * CH
    src3 = jnp.concatenate(
        [src, jnp.zeros((e_pad - e,), jnp.int32)]).reshape(n_rows, CH)
    dst3 = jnp.concatenate(
        [dst, jnp.full((e_pad - e,), n, jnp.int32)]).reshape(n_rows, CH)

    x_pad = jnp.pad(x, ((0, n_pad - n), (0, 0)))
    batch3 = jnp.pad(batch, (0, n_pad - n),
                     constant_values=G).reshape(n_pad // ROWB, 1, ROWB)

    layers = params["layers"]
    l0 = layers[0]
    xl, xr = _mm(x_pad, l0["Wl"][:, _STORE], l0["Wr"][:, _STORE],
                 l0["bl"][_STORE], l0["br"][_STORE])
    for li in range(3):
        slab = _edge_stage(src3, dst3, xl, xr,
                           layers[li]["att"].reshape(HC)[_PERM], n_pad, k0, k1)
        nxt = layers[li + 1]
        xl, xr = _epi_mm(slab, layers[li]["bias"][_PERM],
                         nxt["Wl"][_PERM][:, _STORE], nxt["Wr"][_PERM][:, _STORE],
                         nxt["bl"][_STORE], nxt["br"][_STORE])
    slab = _edge_stage(src3, dst3, xl, xr,
                       layers[3]["att"].reshape(HC)[_PERM], n_pad, k0, k1)
    return _final(slab, layers[3]["bias"][_CPERM], batch3,
                  params["head_w"][_CPERM, :], params["head_b"])
